# Initial kernel scaffold; baseline (speedup 1.0000x reference)
import jax
import jax.numpy as jnp
from jax.experimental import pallas as pl


def kernel(token_ids, polarity_ids, element_ids, gender_ids, token_table, polarity_table, element_table, gender_table):
    raise NotImplementedError("write your pallas kernel here")


# speedup vs baseline: 1.5920x; 1.5920x over previous
import functools
import math

import jax
import jax.numpy as jnp
import numpy as np
from jax import lax
from jax.experimental import pallas as pl
from jax.experimental.pallas import tpu as pltpu
from jax.experimental.pallas import tpu_sc as plsc

VOCAB = 100000
DIM = 128
B = 1024
L = 200
N = B * L

NUM_CORES = 2
NUM_SUBCORES = 16
NW = NUM_CORES * NUM_SUBCORES
LANES = 16

CHUNK = 128
ROWS_PER_W = N // NW
CHUNKS_PER_W = ROWS_PER_W // CHUNK
GROUPS = CHUNK // LANES


def _make_pe() -> np.ndarray:
    position = np.arange(0, L, dtype=np.float32)[:, None]
    div_term = np.exp(
        np.arange(0, DIM, 2, dtype=np.float32) * (-math.log(10000.0) / DIM))
    pe = np.zeros((L, DIM), dtype=np.float32)
    pe[:, 0::2] = np.sin(position * div_term)
    pe[:, 1::2] = np.cos(position * div_term)
    return pe


_PE = _make_pe()


def _body(tok_hbm, pid_hbm, eid_hbm, gid_hbm,
          table_hbm, pol_hbm, elem_hbm, gend_hbm, pe_hbm,
          out_hbm,
          idx_v, pid_v, eid_v, gid_v, rows_v,
          pol_v, elem_v, gend_v, t96_v, pe_v,
          gat_sem):
    wid = lax.axis_index("s") * NUM_CORES + lax.axis_index("c")

    pltpu.sync_copy(pol_hbm, pol_v)
    pltpu.sync_copy(elem_hbm, elem_v)
    pltpu.sync_copy(gend_hbm, gend_v)
    pltpu.sync_copy(pe_hbm, pe_v)

    def build96(s, _):
        p = s // 24
        e = (s % 24) // 4
        g = s % 4
        for c in range(DIM // LANES):
            sl = pl.ds(c * LANES, LANES)
            t96_v[s, sl] = pol_v[p, sl] + elem_v[e, sl] + gend_v[g, sl]
        return ()

    lax.fori_loop(0, 96, build96, (), unroll=False)

    lane_iota = lax.iota(jnp.int32, LANES)

    def do_chunk(ci, _):
        base = (wid * CHUNKS_PER_W + ci) * CHUNK
        crow = wid * CHUNKS_PER_W + ci
        pltpu.sync_copy(tok_hbm.at[crow], idx_v)
        pltpu.sync_copy(pid_hbm.at[crow], pid_v)
        pltpu.sync_copy(eid_hbm.at[crow], eid_v)
        pltpu.sync_copy(gid_hbm.at[crow], gid_v)
        pltpu.async_copy(table_hbm.at[idx_v], rows_v, gat_sem).wait()

        def do_group(rg, _):
            r0 = rg * LANES
            sl = pl.ds(r0, LANES)
            s16 = pid_v[sl] * 24 + eid_v[sl] * 4 + gid_v[sl]
            l16 = lax.rem(base + r0 + lane_iota, L)
            r16 = r0 + lane_iota

            def do_col(c, _):
                c16 = jnp.full((LANES,), c, jnp.int32)
                sm = plsc.load_gather(t96_v, [s16, c16])
                pv = plsc.load_gather(pe_v, [l16, c16])
                plsc.addupdate_scatter(rows_v, [r16, c16], sm + pv)
                return ()

            lax.fori_loop(0, DIM, do_col, (), unroll=4)
            return ()

        lax.fori_loop(0, GROUPS, do_group, (), unroll=False)
        pltpu.sync_copy(rows_v, out_hbm.at[pl.ds(base, CHUNK)])
        return ()

    lax.fori_loop(0, CHUNKS_PER_W, do_chunk, (), unroll=False)


def kernel(token_ids, polarity_ids, element_ids, gender_ids,
           token_table, polarity_table, element_table, gender_table):
    tok = token_ids.reshape(N // CHUNK, CHUNK)
    pid = polarity_ids.reshape(N // CHUNK, CHUNK)
    eid = element_ids.reshape(N // CHUNK, CHUNK)
    gid = gender_ids.reshape(N // CHUNK, CHUNK)
    pe = jnp.asarray(_PE)

    mesh = plsc.VectorSubcoreMesh(
        core_axis_name="c", subcore_axis_name="s",
        num_cores=NUM_CORES, num_subcores=NUM_SUBCORES)

    k = pl.kernel(
        _body,
        out_type=jax.ShapeDtypeStruct((N, DIM), jnp.float32),
        mesh=mesh,
        compiler_params=pltpu.CompilerParams(needs_layout_passes=False),
        scratch_types=[
            pltpu.VMEM((CHUNK,), jnp.int32),
            pltpu.VMEM((CHUNK,), jnp.int32),
            pltpu.VMEM((CHUNK,), jnp.int32),
            pltpu.VMEM((CHUNK,), jnp.int32),
            pltpu.VMEM((CHUNK, DIM), jnp.float32),
            pltpu.VMEM((4, DIM), jnp.float32),
            pltpu.VMEM((6, DIM), jnp.float32),
            pltpu.VMEM((4, DIM), jnp.float32),
            pltpu.VMEM((96, DIM), jnp.float32),
            pltpu.VMEM((L, DIM), jnp.float32),
            pltpu.SemaphoreType.DMA,
        ],
    )
    out = k(tok, pid, eid, gid,
            token_table, polarity_table, element_table, gender_table, pe)
    return out.reshape(B, L, DIM)


# speedup vs baseline: 5.5657x; 3.4960x over previous
import functools
import math

import jax
import jax.numpy as jnp
import numpy as np
from jax import lax
from jax.experimental import pallas as pl
from jax.experimental.pallas import tpu as pltpu
from jax.experimental.pallas import tpu_sc as plsc

VOCAB = 100000
DIM = 128
B = 1024
L = 200
N = B * L

NUM_CORES = 2
NUM_SUBCORES = 16
NW = NUM_CORES * NUM_SUBCORES
LANES = 16

CHUNK = 128
ROWS_PER_W = N // NW
CHUNKS_PER_W = ROWS_PER_W // CHUNK
GROUPS = CHUNK // LANES
CGRP = DIM // LANES


def _make_pe() -> np.ndarray:
    position = np.arange(0, L, dtype=np.float32)[:, None]
    div_term = np.exp(
        np.arange(0, DIM, 2, dtype=np.float32) * (-math.log(10000.0) / DIM))
    pe = np.zeros((L, DIM), dtype=np.float32)
    pe[:, 0::2] = np.sin(position * div_term)
    pe[:, 1::2] = np.cos(position * div_term)
    return pe


_PE = _make_pe()


def _body(tok_hbm, pid_hbm, eid_hbm, gid_hbm,
          table_hbm, pol_hbm, elem_hbm, gend_hbm, pe_hbm,
          out_hbm, t96_hbm,
          idx_v, pid_v, eid_v, gid_v, sidx_v, rows_v, add_v,
          small_v, t96_v, pe_v,
          gat_sem, add_sem):
    sid = lax.axis_index("s")
    wid = sid * NUM_CORES + lax.axis_index("c")

    pltpu.sync_copy(pe_hbm, pe_v)

    @pl.when(sid == 0)
    def _build():
        pltpu.sync_copy(pol_hbm, small_v.at[pl.ds(0, 4)])
        pltpu.sync_copy(elem_hbm, small_v.at[pl.ds(4, 6)])
        pltpu.sync_copy(gend_hbm, small_v.at[pl.ds(10, 4)])

        def build96(s, _):
            p = s // 24
            e = (s % 24) // 4 + 4
            g = s % 4 + 10
            for c in range(CGRP):
                sl = pl.ds(c * LANES, LANES)
                t96_v[s, sl] = small_v[p, sl] + small_v[e, sl] + small_v[g, sl]
            return ()

        lax.fori_loop(0, 96, build96, (), unroll=False)
        pltpu.sync_copy(t96_v, t96_hbm)

    plsc.subcore_barrier()

    lane_iota = lax.iota(jnp.int32, LANES)

    def do_chunk(ci, _):
        crow = wid * CHUNKS_PER_W + ci
        base = crow * CHUNK
        pltpu.sync_copy(tok_hbm.at[crow], idx_v)
        pltpu.sync_copy(pid_hbm.at[crow], pid_v)
        pltpu.sync_copy(eid_hbm.at[crow], eid_v)
        pltpu.sync_copy(gid_hbm.at[crow], gid_v)

        tok_cp = pltpu.async_copy(table_hbm.at[idx_v], rows_v, gat_sem)

        def mk_sidx(rg, _):
            sl = pl.ds(rg * LANES, LANES)
            sidx_v[sl] = pid_v[sl] * 24 + eid_v[sl] * 4 + gid_v[sl]
            return ()

        lax.fori_loop(0, GROUPS, mk_sidx, (), unroll=True)
        add_cp = pltpu.async_copy(t96_hbm.at[sidx_v], add_v, add_sem)
        tok_cp.wait()
        add_cp.wait()

        l0 = lax.rem(base, L)

        def do_row(r, _):
            lr = lax.rem(l0 + r, L)
            for c in range(CGRP):
                sl = pl.ds(c * LANES, LANES)
                plsc.addupdate(rows_v.at[r, sl], add_v[r, sl] + pe_v[lr, sl])
            return ()

        lax.fori_loop(0, CHUNK, do_row, (), unroll=False)
        pltpu.sync_copy(rows_v, out_hbm.at[pl.ds(base, CHUNK)])
        return ()

    lax.fori_loop(0, CHUNKS_PER_W, do_chunk, (), unroll=False)


def kernel(token_ids, polarity_ids, element_ids, gender_ids,
           token_table, polarity_table, element_table, gender_table):
    tok = token_ids.reshape(N // CHUNK, CHUNK)
    pid = polarity_ids.reshape(N // CHUNK, CHUNK)
    eid = element_ids.reshape(N // CHUNK, CHUNK)
    gid = gender_ids.reshape(N // CHUNK, CHUNK)
    pe = jnp.asarray(_PE)

    mesh = plsc.VectorSubcoreMesh(
        core_axis_name="c", subcore_axis_name="s",
        num_cores=NUM_CORES, num_subcores=NUM_SUBCORES)

    k = pl.kernel(
        _body,
        out_type=(jax.ShapeDtypeStruct((N, DIM), jnp.float32),
                  jax.ShapeDtypeStruct((96, DIM), jnp.float32)),
        mesh=mesh,
        compiler_params=pltpu.CompilerParams(needs_layout_passes=False),
        scratch_types=[
            pltpu.VMEM((CHUNK,), jnp.int32),
            pltpu.VMEM((CHUNK,), jnp.int32),
            pltpu.VMEM((CHUNK,), jnp.int32),
            pltpu.VMEM((CHUNK,), jnp.int32),
            pltpu.VMEM((CHUNK,), jnp.int32),
            pltpu.VMEM((CHUNK, DIM), jnp.float32),
            pltpu.VMEM((CHUNK, DIM), jnp.float32),
            pltpu.VMEM((14, DIM), jnp.float32),
            pltpu.VMEM((96, DIM), jnp.float32),
            pltpu.VMEM((L, DIM), jnp.float32),
            pltpu.SemaphoreType.DMA,
            pltpu.SemaphoreType.DMA,
        ],
    )
    out, _ = k(tok, pid, eid, gid,
               token_table, polarity_table, element_table, gender_table, pe)
    return out.reshape(B, L, DIM)


# speedup vs baseline: 7.5844x; 1.3627x over previous
import functools
import math

import jax
import jax.numpy as jnp
import numpy as np
from jax import lax
from jax.experimental import pallas as pl
from jax.experimental.pallas import tpu as pltpu
from jax.experimental.pallas import tpu_sc as plsc

VOCAB = 100000
DIM = 128
B = 1024
L = 200
N = B * L

NUM_CORES = 2
NUM_SUBCORES = 16
NW = NUM_CORES * NUM_SUBCORES
LANES = 16

CHUNK = 128
ROWS_PER_W = N // NW
CHUNKS_PER_W = ROWS_PER_W // CHUNK
GROUPS = CHUNK // LANES
CGRP = DIM // LANES


def _make_pe() -> np.ndarray:
    position = np.arange(0, L, dtype=np.float32)[:, None]
    div_term = np.exp(
        np.arange(0, DIM, 2, dtype=np.float32) * (-math.log(10000.0) / DIM))
    pe = np.zeros((L, DIM), dtype=np.float32)
    pe[:, 0::2] = np.sin(position * div_term)
    pe[:, 1::2] = np.cos(position * div_term)
    return pe


_PE = _make_pe()


def _body(tok_hbm, pid_hbm, eid_hbm, gid_hbm,
          table_hbm, pol_hbm, elem_hbm, gend_hbm, pe_hbm,
          out_hbm, t96_hbm,
          tok0_v, pid0_v, eid0_v, gid0_v, sidx0_v,
          tok1_v, pid1_v, eid1_v, gid1_v, sidx1_v,
          rows0_v, add0_v, rows1_v, add1_v,
          small_v, t96_v, pe_v,
          isem0, isem1, gsem0, gsem1, asem0, asem1, wsem0, wsem1):
    sid = lax.axis_index("s")
    wid = sid * NUM_CORES + lax.axis_index("c")

    idxb = [(tok0_v, pid0_v, eid0_v, gid0_v), (tok1_v, pid1_v, eid1_v, gid1_v)]
    sidx = [sidx0_v, sidx1_v]
    rows = [rows0_v, rows1_v]
    add = [add0_v, add1_v]
    isem = [isem0, isem1]
    gsem = [gsem0, gsem1]
    asem = [asem0, asem1]
    wsem = [wsem0, wsem1]

    pltpu.sync_copy(pe_hbm, pe_v)

    @pl.when(sid == 0)
    def _build():
        pltpu.sync_copy(pol_hbm, small_v.at[pl.ds(0, 4)])
        pltpu.sync_copy(elem_hbm, small_v.at[pl.ds(4, 6)])
        pltpu.sync_copy(gend_hbm, small_v.at[pl.ds(10, 4)])

        def build96(s, _):
            p = s // 24
            e = (s % 24) // 4 + 4
            g = s % 4 + 10
            for c in range(CGRP):
                sl = pl.ds(c * LANES, LANES)
                t96_v[s, sl] = small_v[p, sl] + small_v[e, sl] + small_v[g, sl]
            return ()

        lax.fori_loop(0, 96, build96, (), unroll=False)
        pltpu.sync_copy(t96_v, t96_hbm)

    plsc.subcore_barrier()

    def idx_fetch(c, p):
        crow = wid * CHUNKS_PER_W + c
        pltpu.async_copy(tok_hbm.at[crow], idxb[p][0], isem[p])
        pltpu.async_copy(pid_hbm.at[crow], idxb[p][1], isem[p])
        pltpu.async_copy(eid_hbm.at[crow], idxb[p][2], isem[p])
        pltpu.async_copy(gid_hbm.at[crow], idxb[p][3], isem[p])

    def idx_wait(p):
        for j, src in enumerate((tok_hbm, pid_hbm, eid_hbm, gid_hbm)):
            pltpu.make_async_copy(src.at[0], idxb[p][j], isem[p]).wait()

    def sidx_compute(p):
        tb, pb, eb, gb = idxb[p]
        for rg in range(GROUPS):
            sl = pl.ds(rg * LANES, LANES)
            sidx[p][sl] = pb[sl] * 24 + eb[sl] * 4 + gb[sl]

    def gathers_issue(p):
        pltpu.async_copy(table_hbm.at[idxb[p][0]], rows[p], gsem[p])
        pltpu.async_copy(t96_hbm.at[sidx[p]], add[p], asem[p])

    def gathers_wait(p):
        pltpu.make_async_copy(table_hbm.at[idxb[p][0]], rows[p], gsem[p]).wait()
        pltpu.make_async_copy(t96_hbm.at[sidx[p]], add[p], asem[p]).wait()

    def compute(c, p):
        base = (wid * CHUNKS_PER_W + c) * CHUNK
        l0 = lax.rem(base, L)
        rp, ap = rows[p], add[p]

        def do_row(r, _):
            lr = lax.rem(l0 + r, L)
            for cg in range(CGRP):
                sl = pl.ds(cg * LANES, LANES)
                plsc.addupdate(rp.at[r, sl], ap[r, sl] + pe_v[lr, sl])
            return ()

        lax.fori_loop(0, CHUNK, do_row, (), unroll=False)

    def wb_issue(c, p):
        base = (wid * CHUNKS_PER_W + c) * CHUNK
        pltpu.async_copy(rows[p], out_hbm.at[pl.ds(base, CHUNK)], wsem[p])

    def wb_wait(p):
        pltpu.make_async_copy(
            rows[p], out_hbm.at[pl.ds(0, CHUNK)], wsem[p]).wait()

    idx_fetch(0, 0)
    idx_fetch(1, 1)
    idx_wait(0)
    sidx_compute(0)
    gathers_issue(0)

    def step(i, p):
        q = 1 - p
        gathers_wait(p)

        @pl.when(i + 2 < CHUNKS_PER_W)
        def _():
            idx_fetch(i + 2, p)

        @pl.when(i + 1 < CHUNKS_PER_W)
        def _():
            idx_wait(q)
            sidx_compute(q)

            @pl.when(i >= 1)
            def _():
                wb_wait(q)

            gathers_issue(q)

        compute(i, p)
        wb_issue(i, p)

    def pair(k, _):
        step(2 * k, 0)
        step(2 * k + 1, 1)
        return ()

    lax.fori_loop(0, CHUNKS_PER_W // 2, pair, (), unroll=False)

    wb_wait(0)
    wb_wait(1)


def kernel(token_ids, polarity_ids, element_ids, gender_ids,
           token_table, polarity_table, element_table, gender_table):
    tok = token_ids.reshape(N // CHUNK, CHUNK)
    pid = polarity_ids.reshape(N // CHUNK, CHUNK)
    eid = element_ids.reshape(N // CHUNK, CHUNK)
    gid = gender_ids.reshape(N // CHUNK, CHUNK)
    pe = jnp.asarray(_PE)

    mesh = plsc.VectorSubcoreMesh(
        core_axis_name="c", subcore_axis_name="s",
        num_cores=NUM_CORES, num_subcores=NUM_SUBCORES)

    dma = pltpu.SemaphoreType.DMA
    k = pl.kernel(
        _body,
        out_type=(jax.ShapeDtypeStruct((N, DIM), jnp.float32),
                  jax.ShapeDtypeStruct((96, DIM), jnp.float32)),
        mesh=mesh,
        compiler_params=pltpu.CompilerParams(needs_layout_passes=False),
        scratch_types=[
            pltpu.VMEM((CHUNK,), jnp.int32),
            pltpu.VMEM((CHUNK,), jnp.int32),
            pltpu.VMEM((CHUNK,), jnp.int32),
            pltpu.VMEM((CHUNK,), jnp.int32),
            pltpu.VMEM((CHUNK,), jnp.int32),
            pltpu.VMEM((CHUNK,), jnp.int32),
            pltpu.VMEM((CHUNK,), jnp.int32),
            pltpu.VMEM((CHUNK,), jnp.int32),
            pltpu.VMEM((CHUNK,), jnp.int32),
            pltpu.VMEM((CHUNK,), jnp.int32),
            pltpu.VMEM((CHUNK, DIM), jnp.float32),
            pltpu.VMEM((CHUNK, DIM), jnp.float32),
            pltpu.VMEM((CHUNK, DIM), jnp.float32),
            pltpu.VMEM((CHUNK, DIM), jnp.float32),
            pltpu.VMEM((14, DIM), jnp.float32),
            pltpu.VMEM((96, DIM), jnp.float32),
            pltpu.VMEM((L, DIM), jnp.float32),
            dma, dma, dma, dma, dma, dma, dma, dma,
        ],
    )
    out, _ = k(tok, pid, eid, gid,
               token_table, polarity_table, element_table, gender_table, pe)
    return out.reshape(B, L, DIM)


# speedup vs baseline: 13.1254x; 1.7306x over previous
import functools
import math

import jax
import jax.numpy as jnp
import numpy as np
from jax import lax
from jax.experimental import pallas as pl
from jax.experimental.pallas import tpu as pltpu
from jax.experimental.pallas import tpu_sc as plsc

VOCAB = 100000
DIM = 128
B = 1024
L = 200
N = B * L

NUM_CORES = 2
NUM_SUBCORES = 16
NW = NUM_CORES * NUM_SUBCORES
LANES = 16

CHUNK = 128
ROWS_PER_W = N // NW
CHUNKS_PER_W = ROWS_PER_W // CHUNK
GROUPS = CHUNK // LANES
CGRP = DIM // LANES

NFUSE = 96 * L
FROWS_PER_SUB = NFUSE // NUM_SUBCORES
FBLOCK = 120
FBLOCKS = FROWS_PER_SUB // FBLOCK


def _make_pe() -> np.ndarray:
    position = np.arange(0, L, dtype=np.float32)[:, None]
    div_term = np.exp(
        np.arange(0, DIM, 2, dtype=np.float32) * (-math.log(10000.0) / DIM))
    pe = np.zeros((L, DIM), dtype=np.float32)
    pe[:, 0::2] = np.sin(position * div_term)
    pe[:, 1::2] = np.cos(position * div_term)
    return pe


_PE = _make_pe()


def _body(tok_hbm, pid_hbm, eid_hbm, gid_hbm,
          table_hbm, pol_hbm, elem_hbm, gend_hbm, pe_hbm,
          out_hbm, fuse_hbm,
          tok0_v, pid0_v, eid0_v, gid0_v, sidx0_v,
          tok1_v, pid1_v, eid1_v, gid1_v, sidx1_v,
          rows0_v, add0_v, rows1_v, add1_v,
          small_v, t96_v, pe_v,
          isem0, isem1, gsem0, gsem1, asem0, asem1, wsem0, wsem1):
    sid = lax.axis_index("s")
    wid = sid * NUM_CORES + lax.axis_index("c")

    idxb = [(tok0_v, pid0_v, eid0_v, gid0_v), (tok1_v, pid1_v, eid1_v, gid1_v)]
    sidx = [sidx0_v, sidx1_v]
    rows = [rows0_v, rows1_v]
    add = [add0_v, add1_v]
    isem = [isem0, isem1]
    gsem = [gsem0, gsem1]
    asem = [asem0, asem1]
    wsem = [wsem0, wsem1]

    pltpu.sync_copy(pe_hbm, pe_v)
    pltpu.sync_copy(pol_hbm, small_v.at[pl.ds(0, 4)])
    pltpu.sync_copy(elem_hbm, small_v.at[pl.ds(4, 6)])
    pltpu.sync_copy(gend_hbm, small_v.at[pl.ds(10, 4)])

    def build96(s, _):
        p = s // 24
        e = (s % 24) // 4 + 4
        g = s % 4 + 10
        for c in range(CGRP):
            sl = pl.ds(c * LANES, LANES)
            t96_v[s, sl] = small_v[p, sl] + small_v[e, sl] + small_v[g, sl]
        return ()

    lax.fori_loop(0, 96, build96, (), unroll=False)

    stage = add0_v

    def build_block(bk, _):
        fbase = sid * FROWS_PER_SUB + bk * FBLOCK

        def build_row(r, _):
            f = fbase + r
            s = f // L
            l = f - s * L
            for c in range(CGRP):
                sl = pl.ds(c * LANES, LANES)
                stage[r, sl] = t96_v[s, sl] + pe_v[l, sl]
            return ()

        lax.fori_loop(0, FBLOCK, build_row, (), unroll=False)
        pltpu.sync_copy(stage.at[pl.ds(0, FBLOCK)],
                        fuse_hbm.at[pl.ds(fbase, FBLOCK)])
        return ()

    lax.fori_loop(0, FBLOCKS, build_block, (), unroll=False)
    plsc.subcore_barrier()

    def idx_fetch(c, p):
        crow = wid * CHUNKS_PER_W + c
        pltpu.async_copy(tok_hbm.at[crow], idxb[p][0], isem[p])
        pltpu.async_copy(pid_hbm.at[crow], idxb[p][1], isem[p])
        pltpu.async_copy(eid_hbm.at[crow], idxb[p][2], isem[p])
        pltpu.async_copy(gid_hbm.at[crow], idxb[p][3], isem[p])

    def idx_wait(p):
        for j, src in enumerate((tok_hbm, pid_hbm, eid_hbm, gid_hbm)):
            pltpu.make_async_copy(src.at[0], idxb[p][j], isem[p]).wait()

    lane_iota = lax.iota(jnp.int32, LANES)

    def sidx_compute(c, p):
        base = (wid * CHUNKS_PER_W + c) * CHUNK
        tb, pb, eb, gb = idxb[p]
        for rg in range(GROUPS):
            sl = pl.ds(rg * LANES, LANES)
            l16 = lax.rem(base + rg * LANES + lane_iota, L)
            sidx[p][sl] = (pb[sl] * 24 + eb[sl] * 4 + gb[sl]) * L + l16

    def gathers_issue(p):
        pltpu.async_copy(table_hbm.at[idxb[p][0]], rows[p], gsem[p])
        pltpu.async_copy(fuse_hbm.at[sidx[p]], add[p], asem[p])

    def gathers_wait(p):
        pltpu.make_async_copy(table_hbm.at[idxb[p][0]], rows[p], gsem[p]).wait()
        pltpu.make_async_copy(fuse_hbm.at[sidx[p]], add[p], asem[p]).wait()

    def compute(p):
        rp, ap = rows[p], add[p]

        def do_row(r, _):
            for cg in range(CGRP):
                sl = pl.ds(cg * LANES, LANES)
                plsc.addupdate(rp.at[r, sl], ap[r, sl])
            return ()

        lax.fori_loop(0, CHUNK, do_row, (), unroll=2)

    def wb_issue(c, p):
        base = (wid * CHUNKS_PER_W + c) * CHUNK
        pltpu.async_copy(rows[p], out_hbm.at[pl.ds(base, CHUNK)], wsem[p])

    def wb_wait(p):
        pltpu.make_async_copy(
            rows[p], out_hbm.at[pl.ds(0, CHUNK)], wsem[p]).wait()

    idx_fetch(0, 0)
    idx_fetch(1, 1)
    idx_wait(0)
    sidx_compute(0, 0)
    gathers_issue(0)

    def step(i, p):
        q = 1 - p
        gathers_wait(p)

        @pl.when(i + 2 < CHUNKS_PER_W)
        def _():
            idx_fetch(i + 2, p)

        @pl.when(i + 1 < CHUNKS_PER_W)
        def _():
            idx_wait(q)
            sidx_compute(i + 1, q)

            @pl.when(i >= 1)
            def _():
                wb_wait(q)

            gathers_issue(q)

        compute(p)
        wb_issue(i, p)

    def pair(k, _):
        step(2 * k, 0)
        step(2 * k + 1, 1)
        return ()

    lax.fori_loop(0, CHUNKS_PER_W // 2, pair, (), unroll=False)

    wb_wait(0)
    wb_wait(1)


def kernel(token_ids, polarity_ids, element_ids, gender_ids,
           token_table, polarity_table, element_table, gender_table):
    tok = token_ids.reshape(N // CHUNK, CHUNK)
    pid = polarity_ids.reshape(N // CHUNK, CHUNK)
    eid = element_ids.reshape(N // CHUNK, CHUNK)
    gid = gender_ids.reshape(N // CHUNK, CHUNK)
    pe = jnp.asarray(_PE)

    mesh = plsc.VectorSubcoreMesh(
        core_axis_name="c", subcore_axis_name="s",
        num_cores=NUM_CORES, num_subcores=NUM_SUBCORES)

    dma = pltpu.SemaphoreType.DMA
    k = pl.kernel(
        _body,
        out_type=(jax.ShapeDtypeStruct((N, DIM), jnp.float32),
                  jax.ShapeDtypeStruct((NFUSE, DIM), jnp.float32)),
        mesh=mesh,
        compiler_params=pltpu.CompilerParams(needs_layout_passes=False),
        scratch_types=[
            pltpu.VMEM((CHUNK,), jnp.int32),
            pltpu.VMEM((CHUNK,), jnp.int32),
            pltpu.VMEM((CHUNK,), jnp.int32),
            pltpu.VMEM((CHUNK,), jnp.int32),
            pltpu.VMEM((CHUNK,), jnp.int32),
            pltpu.VMEM((CHUNK,), jnp.int32),
            pltpu.VMEM((CHUNK,), jnp.int32),
            pltpu.VMEM((CHUNK,), jnp.int32),
            pltpu.VMEM((CHUNK,), jnp.int32),
            pltpu.VMEM((CHUNK,), jnp.int32),
            pltpu.VMEM((CHUNK, DIM), jnp.float32),
            pltpu.VMEM((CHUNK, DIM), jnp.float32),
            pltpu.VMEM((CHUNK, DIM), jnp.float32),
            pltpu.VMEM((CHUNK, DIM), jnp.float32),
            pltpu.VMEM((14, DIM), jnp.float32),
            pltpu.VMEM((96, DIM), jnp.float32),
            pltpu.VMEM((L, DIM), jnp.float32),
            dma, dma, dma, dma, dma, dma, dma, dma,
        ],
    )
    out, _ = k(tok, pid, eid, gid,
               token_table, polarity_table, element_table, gender_table, pe)
    return out.reshape(B, L, DIM)
